# baseline (speedup 1.0000x reference)
import jax
import jax.numpy as jnp
from jax import lax
from jax.experimental import pallas as pl
from jax.experimental.pallas import tpu as pltpu
from jax.experimental.pallas import tpu_sc as plsc

N = 10000
E = 320000
D = 128
NC = 2
NS = 16
NW = NC * NS
CH = 128
CPT = 80
HCPT = CPT // 2
NCHUNKS = NW * CPT
E_PAD = NCHUNKS * CH
NP = 10240
RPT = NP // NS


def _fill(buf, n16, val):
    def st(i, c):
        buf[pl.ds(i * 16, 16)] = jnp.full((16,), val, jnp.float32)
        return c
    lax.fori_loop(0, n16, st, 0)


def _deg_body(col2d, deg_out, acc, colv, ones_v, zer_v, sem):
    cid = lax.axis_index("c")
    sid = lax.axis_index("s")
    wid = cid * NS + sid
    _fill(zer_v, RPT // 16, 0.0)
    _fill(ones_v, CH // 16, 1.0)
    pltpu.sync_copy(zer_v, acc.at[pl.ds(sid * RPT, RPT)])
    pltpu.sync_copy(col2d.at[pl.ds(wid * CPT, CPT)], colv)
    plsc.subcore_barrier()

    def fire(i, c):
        pltpu.async_copy(ones_v, acc.at[colv.at[i]], sem, add=True)
        return c
    lax.fori_loop(0, CPT, fire, 0)

    def drain(i, c):
        pltpu.make_async_copy(ones_v, acc.at[colv.at[0]], sem).wait()
        return c
    lax.fori_loop(0, CPT, drain, 0)
    plsc.subcore_barrier()
    pltpu.sync_copy(acc.at[pl.ds(sid * RPT, RPT)],
                    deg_out.at[cid, pl.ds(sid * RPT, RPT)])


def _agg_body(hs, row2d, col2d, out, acc, rowv, colv, d0, d1, sem0, sem1):
    cid = lax.axis_index("c")
    sid = lax.axis_index("s")
    wid = cid * NS + sid
    pltpu.sync_copy(hs.at[pl.ds(sid * RPT, RPT)], acc.at[pl.ds(sid * RPT, RPT)])
    plsc.subcore_barrier()

    def start(i, dbuf, sem):
        pltpu.async_copy(hs.at[rowv.at[i]], dbuf, sem)

    def finish(dbuf, sem):
        pltpu.make_async_copy(hs.at[rowv.at[0]], dbuf, sem).wait()

    def scat(i, dbuf):
        pltpu.sync_copy(dbuf, acc.at[colv.at[i]], add=True)

    for h in range(2):
        pltpu.sync_copy(row2d.at[pl.ds(wid * CPT + h * HCPT, HCPT)], rowv)
        pltpu.sync_copy(col2d.at[pl.ds(wid * CPT + h * HCPT, HCPT)], colv)
        start(0, d0, sem0)

        def body(j, c):
            i0 = 2 * j
            start(i0 + 1, d1, sem1)
            finish(d0, sem0)
            scat(i0, d0)
            start(i0 + 2, d0, sem0)
            finish(d1, sem1)
            scat(i0 + 1, d1)
            return c
        lax.fori_loop(0, HCPT // 2 - 1, body, 0)
        start(HCPT - 1, d1, sem1)
        finish(d0, sem0)
        scat(HCPT - 2, d0)
        finish(d1, sem1)
        scat(HCPT - 1, d1)
    plsc.subcore_barrier()
    pltpu.sync_copy(acc.at[pl.ds(sid * RPT, RPT)],
                    out.at[cid, pl.ds(sid * RPT, RPT)])


def _tc_mm(x_ref, w_ref, h_ref):
    h = jnp.dot(x_ref[...], w_ref[...], preferred_element_type=jnp.float32)
    h_ref[...] = jnp.concatenate(
        [h, jnp.zeros((NP - N, D), jnp.float32)], axis=0)


def _dinv(degt_ref):
    return lax.rsqrt(degt_ref[:, 0:1] + degt_ref[:, 1:2] + 1.0)


def _tc_scale(h_ref, degt_ref, hs_ref):
    hs_ref[...] = h_ref[...] * _dinv(degt_ref)


def _tc_mid(acc_ref, hs_ref, degt_ref, b_ref, w_ref, out_ref):
    dinv = _dinv(degt_ref)
    pre = (acc_ref[0] + acc_ref[1] - hs_ref[...]) * dinv + b_ref[...]
    h1 = jnp.maximum(pre, 0.0)
    out_ref[...] = jnp.dot(h1, w_ref[...],
                           preferred_element_type=jnp.float32) * dinv


def _tc_fin(acc_ref, hs_ref, degt_ref, b_ref, z_ref):
    z = (acc_ref[0] + acc_ref[1] - hs_ref[...]) * _dinv(degt_ref) + b_ref[...]
    z_ref[...] = z[:N]


def kernel(x, edge_index, W1, b1, W2, b2):
    row = edge_index[0].astype(jnp.int32)
    col = edge_index[1].astype(jnp.int32)
    pad = E_PAD - E
    pad_idx = jnp.arange(pad, dtype=jnp.int32) % 128
    row2d = jnp.concatenate([row, pad_idx]).reshape(NCHUNKS, CH)
    col2d = jnp.concatenate([col, N + pad_idx]).reshape(NCHUNKS, CH)

    mesh = plsc.VectorSubcoreMesh(core_axis_name="c", subcore_axis_name="s")
    f32 = jnp.float32

    deg_call = pl.kernel(
        _deg_body,
        out_type=jax.ShapeDtypeStruct((NC, NP), f32),
        mesh=mesh,
        scratch_types=[
            pltpu.VMEM_SHARED((NP,), f32),
            pltpu.VMEM((CPT, CH), jnp.int32),
            pltpu.VMEM((CH,), f32),
            pltpu.VMEM((RPT,), f32),
            pltpu.SemaphoreType.DMA,
        ],
    )
    agg_call = pl.kernel(
        _agg_body,
        out_type=jax.ShapeDtypeStruct((NC, NP, D), f32),
        mesh=mesh,
        scratch_types=[
            pltpu.VMEM_SHARED((NP, D), f32),
            pltpu.VMEM((HCPT, CH), jnp.int32),
            pltpu.VMEM((HCPT, CH), jnp.int32),
            pltpu.VMEM((CH, D), f32),
            pltpu.VMEM((CH, D), f32),
            pltpu.SemaphoreType.DMA,
            pltpu.SemaphoreType.DMA,
        ],
    )
    mm_call = pl.pallas_call(
        _tc_mm,
        out_shape=jax.ShapeDtypeStruct((NP, D), f32),
    )
    scale_call = pl.pallas_call(
        _tc_scale,
        out_shape=jax.ShapeDtypeStruct((NP, D), f32),
    )
    mid_call = pl.pallas_call(
        _tc_mid,
        out_shape=jax.ShapeDtypeStruct((NP, D), f32),
    )
    fin_call = pl.pallas_call(
        _tc_fin,
        out_shape=jax.ShapeDtypeStruct((N, D), f32),
    )

    h1 = mm_call(x, W1)
    deg = deg_call(col2d)
    degt = deg.T
    hs1 = scale_call(h1, degt)
    acc1 = agg_call(hs1, row2d, col2d)
    hs2 = mid_call(acc1, hs1, degt, b1.reshape(1, D), W2)
    acc2 = agg_call(hs2, row2d, col2d)
    z = fin_call(acc2, hs2, degt, b2.reshape(1, D))
    return z


# speedup vs baseline: 1.0122x; 1.0122x over previous
import jax
import jax.numpy as jnp
from jax import lax
from jax.experimental import pallas as pl
from jax.experimental.pallas import tpu as pltpu
from jax.experimental.pallas import tpu_sc as plsc

N = 10000
E = 320000
D = 128
NC = 2
NS = 16
NW = NC * NS
CH = 128
CPT = 80
HCPT = CPT // 2
NCHUNKS = NW * CPT
E_PAD = NCHUNKS * CH
NP = 10240
RPT = NP // NS


def _fill(buf, n16, val):
    def st(i, c):
        buf[pl.ds(i * 16, 16)] = jnp.full((16,), val, jnp.float32)
        return c
    lax.fori_loop(0, n16, st, 0)


def _deg_body(col2d, deg_out, acc, colv, ones_v, zer_v, sem):
    cid = lax.axis_index("c")
    sid = lax.axis_index("s")
    wid = cid * NS + sid
    _fill(zer_v, RPT // 16, 0.0)
    _fill(ones_v, CH // 16, 1.0)
    pltpu.sync_copy(zer_v, acc.at[pl.ds(sid * RPT, RPT)])
    pltpu.sync_copy(col2d.at[pl.ds(wid * CPT, CPT)], colv)
    plsc.subcore_barrier()

    def fire(i, c):
        pltpu.async_copy(ones_v, acc.at[colv.at[i]], sem, add=True)
        return c
    lax.fori_loop(0, CPT, fire, 0)

    def drain(i, c):
        pltpu.make_async_copy(ones_v, acc.at[colv.at[0]], sem).wait()
        return c
    lax.fori_loop(0, CPT, drain, 0)
    plsc.subcore_barrier()
    pltpu.sync_copy(acc.at[pl.ds(sid * RPT, RPT)],
                    deg_out.at[cid, pl.ds(sid * RPT, RPT)])


def _agg_body(hs, row2d, col2d, out, acc, rowv, colv, d0, d1, sem0, sem1):
    cid = lax.axis_index("c")
    sid = lax.axis_index("s")
    wid = cid * NS + sid
    @pl.when(cid == 0)
    def _():
        pltpu.sync_copy(hs.at[pl.ds(sid * RPT, RPT)],
                        acc.at[pl.ds(sid * RPT, RPT)])

    @pl.when(cid != 0)
    def _():
        def zrow(r, c):
            for k in range(D // 16):
                d0[r, pl.ds(k * 16, 16)] = jnp.zeros((16,), jnp.float32)
            return c
        lax.fori_loop(0, CH, zrow, 0)
        for r in range(RPT // CH):
            pltpu.sync_copy(d0, acc.at[pl.ds(sid * RPT + r * CH, CH)])
    plsc.subcore_barrier()

    def start(i, dbuf, sem):
        pltpu.async_copy(hs.at[rowv.at[i]], dbuf, sem)

    def finish(dbuf, sem):
        pltpu.make_async_copy(hs.at[rowv.at[0]], dbuf, sem).wait()

    def scat(i, dbuf):
        pltpu.sync_copy(dbuf, acc.at[colv.at[i]], add=True)

    for h in range(2):
        pltpu.sync_copy(row2d.at[pl.ds(wid * CPT + h * HCPT, HCPT)], rowv)
        pltpu.sync_copy(col2d.at[pl.ds(wid * CPT + h * HCPT, HCPT)], colv)
        start(0, d0, sem0)

        def body(j, c):
            i0 = 2 * j
            start(i0 + 1, d1, sem1)
            finish(d0, sem0)
            scat(i0, d0)
            start(i0 + 2, d0, sem0)
            finish(d1, sem1)
            scat(i0 + 1, d1)
            return c
        lax.fori_loop(0, HCPT // 2 - 1, body, 0)
        start(HCPT - 1, d1, sem1)
        finish(d0, sem0)
        scat(HCPT - 2, d0)
        finish(d1, sem1)
        scat(HCPT - 1, d1)
    plsc.subcore_barrier()
    pltpu.sync_copy(acc.at[pl.ds(sid * RPT, RPT)],
                    out.at[cid, pl.ds(sid * RPT, RPT)])


def _tc_mm(x_ref, w_ref, h_ref):
    h = jnp.dot(x_ref[...], w_ref[...], preferred_element_type=jnp.float32)
    h_ref[...] = jnp.concatenate(
        [h, jnp.zeros((NP - N, D), jnp.float32)], axis=0)


def _dinv(degt_ref):
    return lax.rsqrt(degt_ref[:, 0:1] + degt_ref[:, 1:2] + 1.0)


def _tc_scale(h_ref, degt_ref, hs_ref):
    hs_ref[...] = h_ref[...] * _dinv(degt_ref)


def _tc_mid(acc_ref, degt_ref, b_ref, w_ref, out_ref):
    dinv = _dinv(degt_ref)
    pre = (acc_ref[0] + acc_ref[1]) * dinv + b_ref[...]
    h1 = jnp.maximum(pre, 0.0)
    out_ref[...] = jnp.dot(h1, w_ref[...],
                           preferred_element_type=jnp.float32) * dinv


def _tc_fin(acc_ref, degt_ref, b_ref, z_ref):
    z = (acc_ref[0] + acc_ref[1]) * _dinv(degt_ref) + b_ref[...]
    z_ref[...] = z[:N]


def kernel(x, edge_index, W1, b1, W2, b2):
    row = edge_index[0].astype(jnp.int32)
    col = edge_index[1].astype(jnp.int32)
    pad = E_PAD - E
    pad_idx = jnp.arange(pad, dtype=jnp.int32) % 128
    row2d = jnp.concatenate([row, pad_idx]).reshape(NCHUNKS, CH)
    col2d = jnp.concatenate([col, N + pad_idx]).reshape(NCHUNKS, CH)

    mesh = plsc.VectorSubcoreMesh(core_axis_name="c", subcore_axis_name="s")
    f32 = jnp.float32

    deg_call = pl.kernel(
        _deg_body,
        out_type=jax.ShapeDtypeStruct((NC, NP), f32),
        mesh=mesh,
        scratch_types=[
            pltpu.VMEM_SHARED((NP,), f32),
            pltpu.VMEM((CPT, CH), jnp.int32),
            pltpu.VMEM((CH,), f32),
            pltpu.VMEM((RPT,), f32),
            pltpu.SemaphoreType.DMA,
        ],
    )
    agg_call = pl.kernel(
        _agg_body,
        out_type=jax.ShapeDtypeStruct((NC, NP, D), f32),
        mesh=mesh,
        scratch_types=[
            pltpu.VMEM_SHARED((NP, D), f32),
            pltpu.VMEM((HCPT, CH), jnp.int32),
            pltpu.VMEM((HCPT, CH), jnp.int32),
            pltpu.VMEM((CH, D), f32),
            pltpu.VMEM((CH, D), f32),
            pltpu.SemaphoreType.DMA,
            pltpu.SemaphoreType.DMA,
        ],
    )
    mm_call = pl.pallas_call(
        _tc_mm,
        out_shape=jax.ShapeDtypeStruct((NP, D), f32),
    )
    scale_call = pl.pallas_call(
        _tc_scale,
        out_shape=jax.ShapeDtypeStruct((NP, D), f32),
    )
    mid_call = pl.pallas_call(
        _tc_mid,
        out_shape=jax.ShapeDtypeStruct((NP, D), f32),
    )
    fin_call = pl.pallas_call(
        _tc_fin,
        out_shape=jax.ShapeDtypeStruct((N, D), f32),
    )

    h1 = mm_call(x, W1)
    deg = deg_call(col2d)
    degt = deg.T
    hs1 = scale_call(h1, degt)
    acc1 = agg_call(hs1, row2d, col2d)
    hs2 = mid_call(acc1, degt, b1.reshape(1, D), W2)
    acc2 = agg_call(hs2, row2d, col2d)
    z = fin_call(acc2, degt, b2.reshape(1, D))
    return z
